# Initial kernel scaffold; baseline (speedup 1.0000x reference)
import jax
import jax.numpy as jnp
from jax.experimental import pallas as pl


def kernel(x):
    raise NotImplementedError("write your pallas kernel here")


# speedup vs baseline: 1.3599x; 1.3599x over previous
import functools
from itertools import combinations

import jax
import jax.numpy as jnp
from jax import lax
from jax.experimental import pallas as pl
from jax.experimental.pallas import tpu as pltpu
from jax.experimental.pallas import tpu_sc as plsc

ROWS, COLS = 16384, 16
PAIRS = list(combinations(range(COLS), 2))
TRIPLES = list(combinations(range(COLS), 3))
OUT_COLS = COLS + len(PAIRS) + len(TRIPLES)
PAIR_COL = {c: COLS + i for i, c in enumerate(PAIRS)}
TRIPLE_COL = {c: COLS + len(PAIRS) + i for i, c in enumerate(TRIPLES)}

NC, NS, L = 2, 16, 16
NW = NC * NS
ROWS_PER_W = ROWS // NW
BLK = L
NBLK = ROWS_PER_W // BLK
OPAD = 697


@functools.partial(
    pl.kernel,
    out_type=jax.ShapeDtypeStruct((ROWS, OUT_COLS), jnp.float32),
    mesh=plsc.VectorSubcoreMesh(core_axis_name="c", subcore_axis_name="s"),
    compiler_params=pltpu.CompilerParams(
        use_tc_tiling_on_sc=False, needs_layout_passes=False),
    scratch_types=[
        pltpu.VMEM((BLK, OPAD), jnp.float32),
    ],
)
def _drastic_sc(x_hbm, out_hbm, ov):
    wid = lax.axis_index("s") * NC + lax.axis_index("c")
    row0w = wid * ROWS_PER_W
    rowids = lax.iota(jnp.int32, L)
    zero = jnp.zeros((L,), jnp.float32)

    def body(i, carry):
        row0 = row0w + i * BLK
        pltpu.sync_copy(x_hbm.at[pl.ds(row0, BLK), :], ov.at[:, pl.ds(0, COLS)])
        cols = [
            plsc.load_gather(ov, [rowids, jnp.full((L,), c, jnp.int32)])
            for c in range(COLS)
        ]
        ones = [cols[c] == 1.0 for c in range(COLS)]
        for a, b in PAIRS:
            p = jnp.where(ones[b], cols[a], jnp.where(ones[a], cols[b], zero))
            plsc.store_scatter(
                ov, [rowids, jnp.full((L,), PAIR_COL[(a, b)], jnp.int32)], p)
            p_one = jnp.logical_and(ones[a], ones[b])
            for c in range(b + 1, COLS):
                t = jnp.where(ones[c], p, jnp.where(p_one, cols[c], zero))
                plsc.store_scatter(
                    ov, [rowids, jnp.full((L,), TRIPLE_COL[(a, b, c)], jnp.int32)], t)
        pltpu.sync_copy(ov.at[:, pl.ds(0, OUT_COLS)],
                        out_hbm.at[pl.ds(row0, BLK), :])
        return carry

    lax.fori_loop(0, NBLK, body, 0)


def kernel(x):
    return _drastic_sc(x)
